# baseline (speedup 1.0000x reference)
import jax
import jax.numpy as jnp
import numpy as np
from jax import lax
from jax.experimental import pallas as pl
from jax.experimental.pallas import tpu as pltpu
from jax.experimental.pallas import tpu_sc as plsc

BATCH = 16384
NUM_ROWS = 1000000
EMBED_DIM = 32
PACK = 128 // EMBED_DIM
RESHAPED_N = NUM_ROWS // PACK
NUM_CORES = 2
NUM_SUBCORES = 16
LANES = 16
NUM_WORKERS = NUM_CORES * NUM_SUBCORES
BPW = BATCH // NUM_WORKERS
CHUNK = 128
NCHUNK = BPW // CHUNK


def _sc_body(uid_hbm, iid_hbm, um_hbm, ub_hbm, ib_hbm,
             out_hbm, uidx_v, iidx_v,
             urow0_v, urow1_v, irow0_v, irow1_v, uoff_v, ioff_v,
             umr0_v, umr1_v, imr0_v, imr1_v,
             ub_v, ib_v, out_v, sem0, sem1, bsem):
    wid = lax.axis_index("s") * NUM_CORES + lax.axis_index("c")
    base = wid * BPW

    pltpu.sync_copy(uid_hbm.at[pl.ds(base, BPW)], uidx_v)
    pltpu.sync_copy(iid_hbm.at[pl.ds(base, BPW)], iidx_v)

    cp_ub = pltpu.async_copy(ub_hbm.at[uidx_v], ub_v, bsem)
    cp_ib = pltpu.async_copy(ib_hbm.at[iidx_v], ib_v, bsem)

    urow = (urow0_v, urow1_v)
    irow = (irow0_v, irow1_v)
    umr = (umr0_v, umr1_v)
    imr = (imr0_v, imr1_v)
    sems = (sem0, sem1)

    def fill(c, p):
        def chunk16(t, carry):
            sl_src = pl.ds(c * CHUNK + t * LANES, LANES)
            sl_dst = pl.ds(t * LANES, LANES)
            ju = uidx_v[sl_src]
            urow[p][sl_dst] = ju >> 2
            uoff_v[p, sl_dst] = (ju & 3) << 5
            ji = iidx_v[sl_src]
            irow[p][sl_dst] = RESHAPED_N + (ji >> 2)
            ioff_v[p, sl_dst] = (ji & 3) << 5
            return carry

        lax.fori_loop(0, CHUNK // LANES, chunk16, 0)

    def issue(p):
        pltpu.async_copy(um_hbm.at[urow[p]], umr[p], sems[p])
        pltpu.async_copy(um_hbm.at[irow[p]], imr[p], sems[p])

    def wait(p):
        pltpu.make_async_copy(um_hbm.at[urow[p]], umr[p], sems[p]).wait()
        pltpu.make_async_copy(um_hbm.at[irow[p]], imr[p], sems[p]).wait()

    fill(0, 0)
    issue(0)

    for c in range(NCHUNK):
        p = c % 2
        if c + 1 < NCHUNK:
            fill(c + 1, 1 - p)
            issue(1 - p)
        wait(p)

        def group(t, carry, c=c, p=p):
            sl = pl.ds(t * LANES, LANES)
            k16 = t * LANES + lax.iota(jnp.int32, LANES)
            uoff = uoff_v[p, sl]
            ioff = ioff_v[p, sl]
            acc = jnp.zeros((LANES,), jnp.float32)
            for d in range(EMBED_DIM):
                u = plsc.load_gather(umr[p], [k16, uoff + d])
                i = plsc.load_gather(imr[p], [k16, ioff + d])
                acc = acc + u * i
            out_v[pl.ds(c * CHUNK + t * LANES, LANES)] = acc
            return carry

        lax.fori_loop(0, CHUNK // LANES, group, 0)

    cp_ub.wait()
    cp_ib.wait()

    def add_bias(t, carry):
        sl = pl.ds(t * LANES, LANES)
        out_v[sl] = out_v[sl] + ub_v[sl] + ib_v[sl]
        return carry

    lax.fori_loop(0, BPW // LANES, add_bias, 0)

    pltpu.sync_copy(out_v, out_hbm.at[pl.ds(base, BPW)])


@jax.jit
def _mf_scores(uid, iid, um, ub, ib):
    mesh = plsc.VectorSubcoreMesh(core_axis_name="c", subcore_axis_name="s")
    return pl.kernel(
        _sc_body,
        out_type=jax.ShapeDtypeStruct((BATCH,), jnp.float32),
        mesh=mesh,
        compiler_params=pltpu.CompilerParams(needs_layout_passes=False),
        scratch_types=[
            pltpu.VMEM((BPW,), jnp.int32),
            pltpu.VMEM((BPW,), jnp.int32),
            pltpu.VMEM((CHUNK,), jnp.int32),
            pltpu.VMEM((CHUNK,), jnp.int32),
            pltpu.VMEM((CHUNK,), jnp.int32),
            pltpu.VMEM((CHUNK,), jnp.int32),
            pltpu.VMEM((2, CHUNK), jnp.int32),
            pltpu.VMEM((2, CHUNK), jnp.int32),
            pltpu.VMEM((CHUNK, 128), jnp.float32),
            pltpu.VMEM((CHUNK, 128), jnp.float32),
            pltpu.VMEM((CHUNK, 128), jnp.float32),
            pltpu.VMEM((CHUNK, 128), jnp.float32),
            pltpu.VMEM((BPW,), jnp.float32),
            pltpu.VMEM((BPW,), jnp.float32),
            pltpu.VMEM((BPW,), jnp.float32),
            pltpu.SemaphoreType.DMA,
            pltpu.SemaphoreType.DMA,
            pltpu.SemaphoreType.DMA,
        ],
    )(uid, iid, um, ub, ib)


def kernel(user_ids, item_ids, user_emb, item_emb, user_bias, item_bias):
    uid = user_ids.astype(jnp.int32)
    iid = item_ids.astype(jnp.int32)
    both = jnp.concatenate([user_emb, item_emb], axis=0)
    um = both.reshape(2 * RESHAPED_N, 128)
    return _mf_scores(uid, iid, um,
                      user_bias.reshape(-1), item_bias.reshape(-1))


# speedup vs baseline: 1.2834x; 1.2834x over previous
import jax
import jax.numpy as jnp
import numpy as np
from jax import lax
from jax.experimental import pallas as pl
from jax.experimental.pallas import tpu as pltpu
from jax.experimental.pallas import tpu_sc as plsc

BATCH = 16384
NUM_ROWS = 1000000
EMBED_DIM = 32
PACK = 128 // EMBED_DIM
RESHAPED_N = NUM_ROWS // PACK
NUM_CORES = 2
NUM_SUBCORES = 16
LANES = 16
NUM_WORKERS = NUM_CORES * NUM_SUBCORES
BPW = BATCH // NUM_WORKERS
CHUNK = 128
NCHUNK = BPW // CHUNK


def _sc_body(uid_hbm, iid_hbm, um_hbm, im_hbm, ub_hbm, ib_hbm,
             out_hbm, uidx_v, iidx_v,
             urow0_v, urow1_v, irow0_v, irow1_v, uoff_v, ioff_v,
             umr0_v, umr1_v, imr0_v, imr1_v,
             ub_v, ib_v, out_v, sem0, sem1, bsem):
    wid = lax.axis_index("s") * NUM_CORES + lax.axis_index("c")
    base = wid * BPW

    pltpu.sync_copy(uid_hbm.at[pl.ds(base, BPW)], uidx_v)
    pltpu.sync_copy(iid_hbm.at[pl.ds(base, BPW)], iidx_v)

    cp_ub = pltpu.async_copy(ub_hbm.at[uidx_v], ub_v, bsem)
    cp_ib = pltpu.async_copy(ib_hbm.at[iidx_v], ib_v, bsem)

    urow = (urow0_v, urow1_v)
    irow = (irow0_v, irow1_v)
    umr = (umr0_v, umr1_v)
    imr = (imr0_v, imr1_v)
    sems = (sem0, sem1)

    def fill(c, p):
        def chunk16(t, carry):
            sl_src = pl.ds(c * CHUNK + t * LANES, LANES)
            sl_dst = pl.ds(t * LANES, LANES)
            ju = uidx_v[sl_src]
            urow[p][sl_dst] = ju >> 2
            uoff_v[p, sl_dst] = (ju & 3) << 5
            ji = iidx_v[sl_src]
            irow[p][sl_dst] = ji >> 2
            ioff_v[p, sl_dst] = (ji & 3) << 5
            return carry

        lax.fori_loop(0, CHUNK // LANES, chunk16, 0)

    def issue(p):
        pltpu.async_copy(um_hbm.at[urow[p]], umr[p], sems[p])
        pltpu.async_copy(im_hbm.at[irow[p]], imr[p], sems[p])

    def wait(p):
        pltpu.make_async_copy(um_hbm.at[urow[p]], umr[p], sems[p]).wait()
        pltpu.make_async_copy(im_hbm.at[irow[p]], imr[p], sems[p]).wait()

    fill(0, 0)
    issue(0)

    for c in range(NCHUNK):
        p = c % 2
        if c + 1 < NCHUNK:
            fill(c + 1, 1 - p)
            issue(1 - p)
        wait(p)

        def group(t, carry, c=c, p=p):
            sl = pl.ds(t * LANES, LANES)
            k16 = t * LANES + lax.iota(jnp.int32, LANES)
            uoff = uoff_v[p, sl]
            ioff = ioff_v[p, sl]
            acc = jnp.zeros((LANES,), jnp.float32)
            for d in range(EMBED_DIM):
                u = plsc.load_gather(umr[p], [k16, uoff + d])
                i = plsc.load_gather(imr[p], [k16, ioff + d])
                acc = acc + u * i
            out_v[pl.ds(c * CHUNK + t * LANES, LANES)] = acc
            return carry

        lax.fori_loop(0, CHUNK // LANES, group, 0)

    cp_ub.wait()
    cp_ib.wait()

    def add_bias(t, carry):
        sl = pl.ds(t * LANES, LANES)
        out_v[sl] = out_v[sl] + ub_v[sl] + ib_v[sl]
        return carry

    lax.fori_loop(0, BPW // LANES, add_bias, 0)

    pltpu.sync_copy(out_v, out_hbm.at[pl.ds(base, BPW)])


@jax.jit
def _mf_scores(uid, iid, um, im, ub, ib):
    mesh = plsc.VectorSubcoreMesh(core_axis_name="c", subcore_axis_name="s")
    return pl.kernel(
        _sc_body,
        out_type=jax.ShapeDtypeStruct((BATCH,), jnp.float32),
        mesh=mesh,
        compiler_params=pltpu.CompilerParams(needs_layout_passes=False),
        scratch_types=[
            pltpu.VMEM((BPW,), jnp.int32),
            pltpu.VMEM((BPW,), jnp.int32),
            pltpu.VMEM((CHUNK,), jnp.int32),
            pltpu.VMEM((CHUNK,), jnp.int32),
            pltpu.VMEM((CHUNK,), jnp.int32),
            pltpu.VMEM((CHUNK,), jnp.int32),
            pltpu.VMEM((2, CHUNK), jnp.int32),
            pltpu.VMEM((2, CHUNK), jnp.int32),
            pltpu.VMEM((CHUNK, 128), jnp.float32),
            pltpu.VMEM((CHUNK, 128), jnp.float32),
            pltpu.VMEM((CHUNK, 128), jnp.float32),
            pltpu.VMEM((CHUNK, 128), jnp.float32),
            pltpu.VMEM((BPW,), jnp.float32),
            pltpu.VMEM((BPW,), jnp.float32),
            pltpu.VMEM((BPW,), jnp.float32),
            pltpu.SemaphoreType.DMA,
            pltpu.SemaphoreType.DMA,
            pltpu.SemaphoreType.DMA,
        ],
    )(uid, iid, um, im, ub, ib)


def kernel(user_ids, item_ids, user_emb, item_emb, user_bias, item_bias):
    uid = user_ids.astype(jnp.int32)
    iid = item_ids.astype(jnp.int32)
    um = user_emb.reshape(RESHAPED_N, 128)
    im = item_emb.reshape(RESHAPED_N, 128)
    return _mf_scores(uid, iid, um, im,
                      user_bias.reshape(-1), item_bias.reshape(-1))
